# baseline (speedup 1.0000x reference)
import functools

import jax
import jax.numpy as jnp
from jax import lax
from jax.experimental import pallas as pl
from jax.experimental.pallas import tpu as pltpu
from jax.experimental.pallas import tpu_sc as plsc

_NC = 2
_NS = 16
_NW = _NC * _NS

_D = 32
_BB = 16


def _lookup(x, table):
    b, s = x.shape
    b_per_w = b // _NW
    n_blk = b_per_w // _BB

    mesh = plsc.VectorSubcoreMesh(core_axis_name="c", subcore_axis_name="s")

    @functools.partial(
        pl.kernel,
        out_type=jax.ShapeDtypeStruct((b, s, _D), jnp.float32),
        mesh=mesh,
        scratch_types=[
            pltpu.VMEM((b_per_w, s), jnp.int32),
            pltpu.VMEM((2, _BB, s, _D), jnp.float32),
            pltpu.SemaphoreType.DMA,
            pltpu.SemaphoreType.DMA,
            pltpu.SemaphoreType.DMA,
            pltpu.SemaphoreType.DMA,
        ],
        compiler_params=pltpu.CompilerParams(
            use_tc_tiling_on_sc=False, needs_layout_passes=False
        ),
    )
    def body(x_hbm, table_hbm, out_hbm, idx_v, rows_v, g0, g1, s0, s1):
        wid = lax.axis_index("s") * _NC + lax.axis_index("c")
        pltpu.sync_copy(x_hbm.at[pl.ds(wid * b_per_w, b_per_w)], idx_v)
        out_base = wid * b_per_w

        gsem = (g0, g1)
        ssem = (s0, s1)

        def gather(i):
            buf = i % 2
            copies = []
            for j in range(_BB):
                copies.append(
                    pltpu.async_copy(
                        table_hbm.at[idx_v.at[i * _BB + j]],
                        rows_v.at[buf, j],
                        gsem[buf],
                    )
                )
            return copies

        gd = [None, None]
        sd = [None, None]
        gd[0] = gather(0)
        for i in range(n_blk):
            buf = i % 2
            nbuf = (i + 1) % 2
            if i + 1 < n_blk:
                if sd[nbuf] is not None:
                    sd[nbuf].wait()
                gd[nbuf] = gather(i + 1)
            for c in gd[buf]:
                c.wait()
            sd[buf] = pltpu.async_copy(
                rows_v.at[buf],
                out_hbm.at[pl.ds(out_base + i * _BB, _BB)],
                ssem[buf],
            )
        for d in sd:
            if d is not None:
                d.wait()

    return body(x, table)


def kernel(x, table):
    return _lookup(x.astype(jnp.int32), table)


# speedup vs baseline: 1.0002x; 1.0002x over previous
import functools

import jax
import jax.numpy as jnp
from jax import lax
from jax.experimental import pallas as pl
from jax.experimental.pallas import tpu as pltpu
from jax.experimental.pallas import tpu_sc as plsc

_NC = 2
_NS = 16
_NW = _NC * _NS

_D = 32
_BB = 16


def _lookup(x, table):
    b, s = x.shape
    b_per_w = b // _NW
    n_blk = b_per_w // _BB

    mesh = plsc.VectorSubcoreMesh(core_axis_name="c", subcore_axis_name="s")

    @functools.partial(
        pl.kernel,
        out_type=jax.ShapeDtypeStruct((b, s, _D), jnp.float32),
        mesh=mesh,
        scratch_types=[
            pltpu.VMEM((b_per_w, s), jnp.int32),
            pltpu.VMEM((2, _BB, s, _D), jnp.float32),
            pltpu.SemaphoreType.DMA,
            pltpu.SemaphoreType.DMA,
            pltpu.SemaphoreType.DMA,
            pltpu.SemaphoreType.DMA,
        ],
        compiler_params=pltpu.CompilerParams(use_tc_tiling_on_sc=False),
    )
    def body(x_hbm, table_hbm, out_hbm, idx_v, rows_v, g0, g1, s0, s1):
        wid = lax.axis_index("s") * _NC + lax.axis_index("c")
        pltpu.sync_copy(x_hbm.at[pl.ds(wid * b_per_w, b_per_w)], idx_v)
        out_base = wid * b_per_w

        gsem = (g0, g1)
        ssem = (s0, s1)

        def gather(i):
            buf = i % 2
            copies = []
            for j in range(_BB):
                copies.append(
                    pltpu.async_copy(
                        table_hbm.at[idx_v.at[i * _BB + j]],
                        rows_v.at[buf, j],
                        gsem[buf],
                    )
                )
            return copies

        gd = [None, None]
        sd = [None, None]
        gd[0] = gather(0)
        for i in range(n_blk):
            buf = i % 2
            nbuf = (i + 1) % 2
            if i + 1 < n_blk:
                if sd[nbuf] is not None:
                    sd[nbuf].wait()
                gd[nbuf] = gather(i + 1)
            for c in gd[buf]:
                c.wait()
            sd[buf] = pltpu.async_copy(
                rows_v.at[buf],
                out_hbm.at[pl.ds(out_base + i * _BB, _BB)],
                ssem[buf],
            )
        for d in sd:
            if d is not None:
                d.wait()

    return body(x, table)


def kernel(x, table):
    return _lookup(x.astype(jnp.int32), table)
